# Initial kernel scaffold; baseline (speedup 1.0000x reference)
import jax
import jax.numpy as jnp
from jax import lax
from jax.experimental import pallas as pl
from jax.experimental.pallas import tpu as pltpu
from jax.experimental.pallas import tpu_sc as plsc

_N = 10000
_E = 320000
_DF = 128
_DH = 16
_DO = 2

_NC = 2
_NS = 16
_NW = _NC * _NS
_K = 128
_CH = 79
_EPT = _K * _CH
_EPAD = _EPT * _NW
_NPAD = 10016
_ZR = _NPAD // _NS
_OR = _N // _NS

_mesh = plsc.VectorSubcoreMesh(core_axis_name="c", subcore_axis_name="s")


def _sc_propagate_body(table_hbm, src_hbm, dst_hbm, zeros_hbm, out_hbm,
                       sidx, didx, rows, acc, gsem):
    c = lax.axis_index("c")
    s = lax.axis_index("s")
    pltpu.sync_copy(zeros_hbm.at[pl.ds(s * _ZR, _ZR)],
                    acc.at[pl.ds(s * _ZR, _ZR)])
    plsc.subcore_barrier()
    base = (c * _NS + s) * _EPT

    @pl.loop(0, _CH)
    def _(i):
        off = base + i * _K
        pltpu.sync_copy(src_hbm.at[pl.ds(off, _K)], sidx)
        pltpu.sync_copy(dst_hbm.at[pl.ds(off, _K)], didx)
        pltpu.async_copy(table_hbm.at[sidx], rows, gsem).wait()
        pltpu.sync_copy(rows, acc.at[didx], add=True)

    plsc.subcore_barrier()
    pltpu.sync_copy(acc.at[pl.ds(s * _OR, _OR)],
                    out_hbm.at[c, pl.ds(s * _OR, _OR)])


_sc_propagate = pl.kernel(
    _sc_propagate_body,
    out_type=jax.ShapeDtypeStruct((_NC, _N, _DH), jnp.float32),
    mesh=_mesh,
    scratch_types=[
        pltpu.VMEM((_K,), jnp.int32),
        pltpu.VMEM((_K,), jnp.int32),
        pltpu.VMEM((_K, _DH), jnp.float32),
        pltpu.VMEM_SHARED((_NPAD, _DH), jnp.float32),
        pltpu.SemaphoreType.DMA,
    ],
)


def _mm1_body(x_ref, w_ref, o_ref):
    o_ref[...] = jnp.dot(x_ref[...], w_ref[...],
                         preferred_element_type=jnp.float32)


_tc_mm1 = pl.pallas_call(
    _mm1_body,
    out_shape=jax.ShapeDtypeStruct((_N, _DH), jnp.float32),
)


def _scale_body(degp_ref, h1_ref, tab_ref, dinv_ref):
    dinv = lax.rsqrt(degp_ref[0] + degp_ref[1] + 1.0)
    dinv_ref[...] = dinv
    tab_ref[...] = jnp.zeros((_NPAD, _DH), jnp.float32)
    tab_ref[0:_N, :] = dinv * h1_ref[...]


_tc_scale = pl.pallas_call(
    _scale_body,
    out_shape=(
        jax.ShapeDtypeStruct((_NPAD, _DH), jnp.float32),
        jax.ShapeDtypeStruct((_N, _DH), jnp.float32),
    ),
)


def _layer1_body(p_ref, u_ref, dinv_ref, b_ref, v_ref):
    p = p_ref[0] + p_ref[1] + u_ref[0:_N, :]
    h = jnp.maximum(dinv_ref[...] * p + b_ref[...], 0.0)
    v_ref[...] = jnp.zeros((_NPAD, _DH), jnp.float32)
    v_ref[0:_N, :] = dinv_ref[...] * h


_tc_layer1 = pl.pallas_call(
    _layer1_body,
    out_shape=jax.ShapeDtypeStruct((_NPAD, _DH), jnp.float32),
)


def _final_body(p_ref, v_ref, dinv_ref, w2_ref, b2_ref, o_ref):
    g = dinv_ref[...] * (p_ref[0] + p_ref[1] + v_ref[0:_N, :])
    z = jnp.dot(g, w2_ref[...], preferred_element_type=jnp.float32) + b2_ref[...]
    m = jnp.max(z, axis=1, keepdims=True)
    o_ref[...] = (z - m) - jnp.log(jnp.sum(jnp.exp(z - m), axis=1,
                                           keepdims=True))


_tc_final = pl.pallas_call(
    _final_body,
    out_shape=jax.ShapeDtypeStruct((_N, _DO), jnp.float32),
)


def kernel(x, edge_index, W1, b1, W2, b2):
    src = edge_index[0].astype(jnp.int32)
    dst = edge_index[1].astype(jnp.int32)
    pad = jnp.full((_EPAD - _E,), _N, jnp.int32)
    src_pad = jnp.concatenate([src, pad])
    dst_pad = jnp.concatenate([dst, pad])
    zeros_tab = jnp.zeros((_NPAD, _DH), jnp.float32)
    ones_tab = jnp.ones((_NPAD, _DH), jnp.float32)

    deg_parts = _sc_propagate(ones_tab, src_pad, dst_pad, zeros_tab)
    h1 = _tc_mm1(x, W1)
    u_tab, dinv = _tc_scale(deg_parts, h1)
    p1 = _sc_propagate(u_tab, src_pad, dst_pad, zeros_tab)
    v_tab = _tc_layer1(p1, u_tab, dinv, b1.reshape(1, _DH))
    p2 = _sc_propagate(v_tab, src_pad, dst_pad, zeros_tab)
    return _tc_final(p2, v_tab, dinv, W2, b2.reshape(1, _DO))


# speedup vs baseline: 16.6109x; 16.6109x over previous
import jax
import jax.numpy as jnp
from jax import lax
from jax.experimental import pallas as pl
from jax.experimental.pallas import tpu as pltpu
from jax.experimental.pallas import tpu_sc as plsc

_N = 10000
_E = 320000
_DF = 128
_DH = 16
_DO = 2

_NC = 2
_NS = 16
_NW = _NC * _NS
_K = 128
_CH = 79
_EPT = _K * _CH
_EPAD = _EPT * _NW
_NPAD = 10112
_ZR = _NPAD // _NS

def _sc_propagate_body(table_hbm, src_hbm, dst_hbm, zeros_hbm, out_hbm,
                       sidx, didx, rows, acc, gsem):
    c = lax.axis_index("c")
    s = lax.axis_index("s")
    pltpu.sync_copy(zeros_hbm.at[pl.ds(s * _ZR, _ZR)],
                    acc.at[pl.ds(s * _ZR, _ZR)])
    plsc.subcore_barrier()
    base = (c * _NS + s) * _EPT

    @pl.loop(0, _CH)
    def _(i):
        off = base + i * _K
        pltpu.sync_copy(src_hbm.at[pl.ds(off, _K)], sidx)
        pltpu.sync_copy(dst_hbm.at[pl.ds(off, _K)], didx)
        pltpu.async_copy(table_hbm.at[sidx], rows, gsem).wait()
        pltpu.sync_copy(rows, acc.at[didx], add=True)

    plsc.subcore_barrier()
    pltpu.sync_copy(acc.at[pl.ds(s * _ZR, _ZR)],
                    out_hbm.at[c, pl.ds(s * _ZR, _ZR)])


_SC_PROP_CACHE = []


def _sc_propagate(table, src, dst, zeros):
    if not _SC_PROP_CACHE:
        mesh = plsc.VectorSubcoreMesh(core_axis_name="c", subcore_axis_name="s")
        _SC_PROP_CACHE.append(pl.kernel(
            _sc_propagate_body,
            out_type=jax.ShapeDtypeStruct((_NC, _NPAD, _DH), jnp.float32),
            mesh=mesh,
            compiler_params=pltpu.CompilerParams(use_tc_tiling_on_sc=False),
            scratch_types=[
                pltpu.VMEM((_K,), jnp.int32),
                pltpu.VMEM((_K,), jnp.int32),
                pltpu.VMEM((_K, _DH), jnp.float32),
                pltpu.VMEM_SHARED((_NPAD, _DH), jnp.float32),
                pltpu.SemaphoreType.DMA,
            ],
        ))
    return _SC_PROP_CACHE[0](table, src, dst, zeros)


def _mm1_body(x_ref, w_ref, o_ref):
    o_ref[...] = jnp.dot(x_ref[...], w_ref[...],
                         preferred_element_type=jnp.float32)


_tc_mm1 = pl.pallas_call(
    _mm1_body,
    out_shape=jax.ShapeDtypeStruct((_N, _DH), jnp.float32),
)


def _scale_body(degp_ref, h1_ref, tab_ref, dinv_ref):
    dinv = lax.rsqrt(degp_ref[0, 0:_N, :] + degp_ref[1, 0:_N, :] + 1.0)
    dinv_ref[...] = dinv
    tab_ref[...] = jnp.zeros((_NPAD, _DH), jnp.float32)
    tab_ref[0:_N, :] = dinv * h1_ref[...]


_tc_scale = pl.pallas_call(
    _scale_body,
    out_shape=(
        jax.ShapeDtypeStruct((_NPAD, _DH), jnp.float32),
        jax.ShapeDtypeStruct((_N, _DH), jnp.float32),
    ),
)


def _layer1_body(p_ref, u_ref, dinv_ref, b_ref, v_ref):
    p = p_ref[0, 0:_N, :] + p_ref[1, 0:_N, :] + u_ref[0:_N, :]
    h = jnp.maximum(dinv_ref[...] * p + b_ref[...], 0.0)
    v_ref[...] = jnp.zeros((_NPAD, _DH), jnp.float32)
    v_ref[0:_N, :] = dinv_ref[...] * h


_tc_layer1 = pl.pallas_call(
    _layer1_body,
    out_shape=jax.ShapeDtypeStruct((_NPAD, _DH), jnp.float32),
)


def _final_body(p_ref, v_ref, dinv_ref, w2_ref, b2_ref, o_ref):
    g = dinv_ref[...] * (p_ref[0, 0:_N, :] + p_ref[1, 0:_N, :]
                         + v_ref[0:_N, :])
    z = jnp.dot(g, w2_ref[...], preferred_element_type=jnp.float32) + b2_ref[...]
    m = jnp.max(z, axis=1, keepdims=True)
    o_ref[...] = (z - m) - jnp.log(jnp.sum(jnp.exp(z - m), axis=1,
                                           keepdims=True))


_tc_final = pl.pallas_call(
    _final_body,
    out_shape=jax.ShapeDtypeStruct((_N, _DO), jnp.float32),
)


def kernel(x, edge_index, W1, b1, W2, b2):
    src = edge_index[0].astype(jnp.int32)
    dst = edge_index[1].astype(jnp.int32)
    pad = jnp.full((_EPAD - _E,), _N, jnp.int32)
    src_pad = jnp.concatenate([src, pad])
    dst_pad = jnp.concatenate([dst, pad])
    zeros_tab = jnp.zeros((_NPAD, _DH), jnp.float32)
    ones_tab = jnp.ones((_NPAD, _DH), jnp.float32)

    deg_parts = _sc_propagate(ones_tab, src_pad, dst_pad, zeros_tab)
    h1 = _tc_mm1(x, W1)
    u_tab, dinv = _tc_scale(deg_parts, h1)
    p1 = _sc_propagate(u_tab, src_pad, dst_pad, zeros_tab)
    v_tab = _tc_layer1(p1, u_tab, dinv, b1.reshape(1, _DH))
    p2 = _sc_propagate(v_tab, src_pad, dst_pad, zeros_tab)
    return _tc_final(p2, v_tab, dinv, W2, b2.reshape(1, _DO))


# speedup vs baseline: 31.4926x; 1.8959x over previous
import jax
import jax.numpy as jnp
from jax import lax
from jax.experimental import pallas as pl
from jax.experimental.pallas import tpu as pltpu
from jax.experimental.pallas import tpu_sc as plsc

_N = 10000
_E = 320000
_DF = 128
_DH = 16
_DO = 2

_NC = 2
_NS = 16
_NW = _NC * _NS
_K = 128
_CH = 80
_EPT = _K * _CH
_EPAD = _EPT * _NW
_NCH = _NW * _CH
_NPAD = 10112
_ZR = _NPAD // _NS

def _sc_propagate_body(table_hbm, edges_hbm, zeros_hbm, out_hbm,
                       i0, i1, i2, i3, r0, r1, r2, r3, acc,
                       si0, si1, si2, si3, sg0, sg1, sg2, sg3):
    c = lax.axis_index("c")
    s = lax.axis_index("s")
    pltpu.sync_copy(zeros_hbm.at[pl.ds(s * _ZR, _ZR)],
                    acc.at[pl.ds(s * _ZR, _ZR)])
    plsc.subcore_barrier()
    g0 = (c * _NS + s) * _CH
    idx = (i0, i1, i2, i3)
    rows = (r0, r1, r2, r3)
    isem = (si0, si1, si2, si3)
    gsem = (sg0, sg1, sg2, sg3)

    def idx_start(j, p):
        pltpu.async_copy(edges_hbm.at[g0 + j], idx[p], isem[p])

    def idx_wait(p):
        pltpu.make_async_copy(edges_hbm.at[g0], idx[p], isem[p]).wait()

    def gather_start(p):
        pltpu.async_copy(table_hbm.at[idx[p].at[0]], rows[p], gsem[p])

    def gather_wait(p):
        pltpu.make_async_copy(table_hbm.at[idx[p].at[0]], rows[p],
                              gsem[p]).wait()

    def scatter(p):
        pltpu.sync_copy(rows[p], acc.at[idx[p].at[1]], add=True)

    for b in range(4):
        idx_start(b, b)
    for b in range(2):
        idx_wait(b)
        gather_start(b)

    @pl.loop(0, _CH - 4, step=4)
    def _(i):
        for b in range(4):
            q = (b + 2) % 4
            gather_wait(b)
            scatter(b)
            idx_wait(q)
            gather_start(q)
            idx_start(i + b + 4, b)

    for b in range(2):
        gather_wait(b)
        scatter(b)
        idx_wait(b + 2)
        gather_start(b + 2)
    for b in range(2, 4):
        gather_wait(b)
        scatter(b)

    plsc.subcore_barrier()
    pltpu.sync_copy(acc.at[pl.ds(s * _ZR, _ZR)],
                    out_hbm.at[c, pl.ds(s * _ZR, _ZR)])


_SC_PROP_CACHE = []


def _sc_propagate(table, edges, zeros):
    if not _SC_PROP_CACHE:
        mesh = plsc.VectorSubcoreMesh(core_axis_name="c", subcore_axis_name="s")
        _SC_PROP_CACHE.append(pl.kernel(
            _sc_propagate_body,
            out_type=jax.ShapeDtypeStruct((_NC, _NPAD, _DH), jnp.float32),
            mesh=mesh,
            compiler_params=pltpu.CompilerParams(use_tc_tiling_on_sc=False),
            scratch_types=(
                [pltpu.VMEM((2, _K), jnp.int32)] * 4
                + [pltpu.VMEM((_K, _DH), jnp.float32)] * 4
                + [pltpu.VMEM_SHARED((_NPAD, _DH), jnp.float32)]
                + [pltpu.SemaphoreType.DMA] * 8
            ),
        ))
    return _SC_PROP_CACHE[0](table, edges, zeros)


def _mm1_body(x_ref, w_ref, o_ref):
    o_ref[...] = jnp.dot(x_ref[...], w_ref[...],
                         preferred_element_type=jnp.float32)


_tc_mm1 = pl.pallas_call(
    _mm1_body,
    out_shape=jax.ShapeDtypeStruct((_N, _DH), jnp.float32),
)


def _scale_body(degp_ref, h1_ref, tab_ref, dinv_ref):
    dinv = lax.rsqrt(degp_ref[0, 0:_N, :] + degp_ref[1, 0:_N, :] + 1.0)
    dinv_ref[...] = dinv
    tab_ref[...] = jnp.zeros((_NPAD, _DH), jnp.float32)
    tab_ref[0:_N, :] = dinv * h1_ref[...]


_tc_scale = pl.pallas_call(
    _scale_body,
    out_shape=(
        jax.ShapeDtypeStruct((_NPAD, _DH), jnp.float32),
        jax.ShapeDtypeStruct((_N, _DH), jnp.float32),
    ),
)


def _layer1_body(p_ref, u_ref, dinv_ref, b_ref, v_ref):
    p = p_ref[0, 0:_N, :] + p_ref[1, 0:_N, :] + u_ref[0:_N, :]
    h = jnp.maximum(dinv_ref[...] * p + b_ref[...], 0.0)
    v_ref[...] = jnp.zeros((_NPAD, _DH), jnp.float32)
    v_ref[0:_N, :] = dinv_ref[...] * h


_tc_layer1 = pl.pallas_call(
    _layer1_body,
    out_shape=jax.ShapeDtypeStruct((_NPAD, _DH), jnp.float32),
)


def _final_body(p_ref, v_ref, dinv_ref, w2_ref, b2_ref, o_ref):
    g = dinv_ref[...] * (p_ref[0, 0:_N, :] + p_ref[1, 0:_N, :]
                         + v_ref[0:_N, :])
    z = jnp.dot(g, w2_ref[...], preferred_element_type=jnp.float32) + b2_ref[...]
    m = jnp.max(z, axis=1, keepdims=True)
    o_ref[...] = (z - m) - jnp.log(jnp.sum(jnp.exp(z - m), axis=1,
                                           keepdims=True))


_tc_final = pl.pallas_call(
    _final_body,
    out_shape=jax.ShapeDtypeStruct((_N, _DO), jnp.float32),
)


def kernel(x, edge_index, W1, b1, W2, b2):
    src = edge_index[0].astype(jnp.int32)
    dst = edge_index[1].astype(jnp.int32)
    pad = jnp.full((_EPAD - _E,), _N, jnp.int32)
    src_pad = jnp.concatenate([src, pad]).reshape(_NCH, _K)
    dst_pad = jnp.concatenate([dst, pad]).reshape(_NCH, _K)
    edges = jnp.stack([src_pad, dst_pad], axis=1)
    zeros_tab = jnp.zeros((_NPAD, _DH), jnp.float32)
    ones_tab = jnp.ones((_NPAD, _DH), jnp.float32)

    deg_parts = _sc_propagate(ones_tab, edges, zeros_tab)
    h1 = _tc_mm1(x, W1)
    u_tab, dinv = _tc_scale(deg_parts, h1)
    p1 = _sc_propagate(u_tab, edges, zeros_tab)
    v_tab = _tc_layer1(p1, u_tab, dinv, b1.reshape(1, _DH))
    p2 = _sc_propagate(v_tab, edges, zeros_tab)
    return _tc_final(p2, v_tab, dinv, W2, b2.reshape(1, _DO))


# speedup vs baseline: 31.7264x; 1.0074x over previous
import jax
import jax.numpy as jnp
from jax import lax
from jax.experimental import pallas as pl
from jax.experimental.pallas import tpu as pltpu
from jax.experimental.pallas import tpu_sc as plsc

_N = 10000
_E = 320000
_DF = 128
_DH = 16
_DO = 2

_NC = 2
_NS = 16
_NW = _NC * _NS
_K = 128
_CH = 80
_EPT = _K * _CH
_EPAD = _EPT * _NW
_NCH = _NW * _CH
_NPAD = 10112
_ZR = _NPAD // _NS

_D = 8


def _sc_propagate_body(table_hbm, edges_hbm, zeros_hbm, out_hbm, *refs):
    idx = refs[0:_D]
    rows = refs[_D:2 * _D]
    acc = refs[2 * _D]
    isem = refs[2 * _D + 1:2 * _D + 1 + _D]
    gsem = refs[2 * _D + 1 + _D:2 * _D + 1 + 2 * _D]
    ssem = refs[2 * _D + 1 + 2 * _D:2 * _D + 1 + 3 * _D]
    c = lax.axis_index("c")
    s = lax.axis_index("s")
    pltpu.sync_copy(zeros_hbm.at[pl.ds(s * _ZR, _ZR)],
                    acc.at[pl.ds(s * _ZR, _ZR)])
    plsc.subcore_barrier()
    g0 = (c * _NS + s) * _CH

    def idx_start(j, p):
        pltpu.async_copy(edges_hbm.at[g0 + j], idx[p], isem[p])

    def idx_wait(p):
        pltpu.make_async_copy(edges_hbm.at[g0], idx[p], isem[p]).wait()

    def gather_start(p):
        pltpu.async_copy(table_hbm.at[idx[p].at[0]], rows[p], gsem[p])

    def gather_wait(p):
        pltpu.make_async_copy(table_hbm.at[idx[p].at[0]], rows[p],
                              gsem[p]).wait()

    def scatter_start(p):
        pltpu.async_copy(rows[p], acc.at[idx[p].at[1]], ssem[p], add=True)

    def scatter_wait(p):
        pltpu.make_async_copy(rows[p], acc.at[idx[p].at[1]], ssem[p]).wait()

    def item(j, p, first=False, last=False):
        gather_wait(p)
        scatter_start(p)
        if not first:
            scatter_wait((p - 4) % _D)
        if not last:
            idx_start(j + 4, (p + 4) % _D)
            idx_wait((p + 2) % _D)
            gather_start((p + 2) % _D)

    for b in range(4):
        idx_start(b, b)
    for b in range(2):
        idx_wait(b)
        gather_start(b)
    for b in range(_D):
        item(b, b, first=(b < 4))

    @pl.loop(_D, _CH - _D, step=_D)
    def _(i):
        for b in range(_D):
            item(i + b, b)

    for b in range(_D):
        j = _CH - _D + b
        gather_wait(b)
        scatter_start(b)
        scatter_wait((b - 4) % _D)
        if j + 4 < _CH:
            idx_start(j + 4, (b + 4) % _D)
        if j + 2 < _CH:
            idx_wait((b + 2) % _D)
            gather_start((b + 2) % _D)
    for b in range(4, _D):
        scatter_wait(b)

    plsc.subcore_barrier()
    pltpu.sync_copy(acc.at[pl.ds(s * _ZR, _ZR)],
                    out_hbm.at[c, pl.ds(s * _ZR, _ZR)])


_SC_PROP_CACHE = []


def _sc_propagate(table, edges, zeros):
    if not _SC_PROP_CACHE:
        mesh = plsc.VectorSubcoreMesh(core_axis_name="c", subcore_axis_name="s")
        _SC_PROP_CACHE.append(pl.kernel(
            _sc_propagate_body,
            out_type=jax.ShapeDtypeStruct((_NC, _NPAD, _DH), jnp.float32),
            mesh=mesh,
            compiler_params=pltpu.CompilerParams(use_tc_tiling_on_sc=False),
            scratch_types=(
                [pltpu.VMEM((2, _K), jnp.int32)] * _D
                + [pltpu.VMEM((_K, _DH), jnp.float32)] * _D
                + [pltpu.VMEM_SHARED((_NPAD, _DH), jnp.float32)]
                + [pltpu.SemaphoreType.DMA] * (3 * _D)
            ),
        ))
    return _SC_PROP_CACHE[0](table, edges, zeros)


def _mm1_body(x_ref, w_ref, o_ref):
    o_ref[...] = jnp.dot(x_ref[...], w_ref[...],
                         preferred_element_type=jnp.float32)


_tc_mm1 = pl.pallas_call(
    _mm1_body,
    out_shape=jax.ShapeDtypeStruct((_N, _DH), jnp.float32),
)


def _scale_body(degp_ref, h1_ref, tab_ref, dinv_ref):
    dinv = lax.rsqrt(degp_ref[0, 0:_N, :] + degp_ref[1, 0:_N, :] + 1.0)
    dinv_ref[...] = dinv
    tab_ref[...] = jnp.zeros((_NPAD, _DH), jnp.float32)
    tab_ref[0:_N, :] = dinv * h1_ref[...]


_tc_scale = pl.pallas_call(
    _scale_body,
    out_shape=(
        jax.ShapeDtypeStruct((_NPAD, _DH), jnp.float32),
        jax.ShapeDtypeStruct((_N, _DH), jnp.float32),
    ),
)


def _layer1_body(p_ref, u_ref, dinv_ref, b_ref, v_ref):
    p = p_ref[0, 0:_N, :] + p_ref[1, 0:_N, :] + u_ref[0:_N, :]
    h = jnp.maximum(dinv_ref[...] * p + b_ref[...], 0.0)
    v_ref[...] = jnp.zeros((_NPAD, _DH), jnp.float32)
    v_ref[0:_N, :] = dinv_ref[...] * h


_tc_layer1 = pl.pallas_call(
    _layer1_body,
    out_shape=jax.ShapeDtypeStruct((_NPAD, _DH), jnp.float32),
)


def _final_body(p_ref, v_ref, dinv_ref, w2_ref, b2_ref, o_ref):
    g = dinv_ref[...] * (p_ref[0, 0:_N, :] + p_ref[1, 0:_N, :]
                         + v_ref[0:_N, :])
    z = jnp.dot(g, w2_ref[...], preferred_element_type=jnp.float32) + b2_ref[...]
    m = jnp.max(z, axis=1, keepdims=True)
    o_ref[...] = (z - m) - jnp.log(jnp.sum(jnp.exp(z - m), axis=1,
                                           keepdims=True))


_tc_final = pl.pallas_call(
    _final_body,
    out_shape=jax.ShapeDtypeStruct((_N, _DO), jnp.float32),
)


def kernel(x, edge_index, W1, b1, W2, b2):
    src = edge_index[0].astype(jnp.int32)
    dst = edge_index[1].astype(jnp.int32)
    pad = jnp.full((_EPAD - _E,), _N, jnp.int32)
    src_pad = jnp.concatenate([src, pad]).reshape(_NCH, _K)
    dst_pad = jnp.concatenate([dst, pad]).reshape(_NCH, _K)
    edges = jnp.stack([src_pad, dst_pad], axis=1)
    zeros_tab = jnp.zeros((_NPAD, _DH), jnp.float32)
    ones_tab = jnp.ones((_NPAD, _DH), jnp.float32)

    deg_parts = _sc_propagate(ones_tab, edges, zeros_tab)
    h1 = _tc_mm1(x, W1)
    u_tab, dinv = _tc_scale(deg_parts, h1)
    p1 = _sc_propagate(u_tab, edges, zeros_tab)
    v_tab = _tc_layer1(p1, u_tab, dinv, b1.reshape(1, _DH))
    p2 = _sc_propagate(v_tab, edges, zeros_tab)
    return _tc_final(p2, v_tab, dinv, W2, b2.reshape(1, _DO))


# speedup vs baseline: 56.1964x; 1.7713x over previous
import jax
import jax.numpy as jnp
from jax import lax
from jax.experimental import pallas as pl
from jax.experimental.pallas import tpu as pltpu
from jax.experimental.pallas import tpu_sc as plsc

_N = 10000
_E = 320000
_DF = 128
_DH = 16
_DO = 2

_NC = 2
_NS = 16
_NW = _NC * _NS
_K = 128
_CH = 80
_EPT = _K * _CH
_EPAD = _EPT * _NW
_NCH = _NW * _CH
_NPAD = 10112
_ZR = _NPAD // _NS

_D = 8


def _sc_propagate_body(table_hbm, edges_hbm, zeros_hbm, out_hbm, *refs):
    idx = refs[0:_D]
    rows = refs[_D:2 * _D]
    acc = refs[2 * _D]
    tab = refs[2 * _D + 1]
    isem = refs[2 * _D + 2:2 * _D + 2 + _D]
    gsem = refs[2 * _D + 2 + _D:2 * _D + 2 + 2 * _D]
    ssem = refs[2 * _D + 2 + 2 * _D:2 * _D + 2 + 3 * _D]
    c = lax.axis_index("c")
    s = lax.axis_index("s")
    pltpu.sync_copy(table_hbm.at[pl.ds(s * _ZR, _ZR)],
                    tab.at[pl.ds(s * _ZR, _ZR)])
    pltpu.sync_copy(zeros_hbm.at[pl.ds(s * _ZR, _ZR)],
                    acc.at[pl.ds(s * _ZR, _ZR)])
    plsc.subcore_barrier()
    g0 = (c * _NS + s) * _CH

    def idx_start(j, p):
        pltpu.async_copy(edges_hbm.at[g0 + j], idx[p], isem[p])

    def idx_wait(p):
        pltpu.make_async_copy(edges_hbm.at[g0], idx[p], isem[p]).wait()

    def gather_start(p):
        pltpu.async_copy(tab.at[idx[p].at[0]], rows[p], gsem[p])

    def gather_wait(p):
        pltpu.make_async_copy(tab.at[idx[p].at[0]], rows[p],
                              gsem[p]).wait()

    def scatter_start(p):
        pltpu.async_copy(rows[p], acc.at[idx[p].at[1]], ssem[p], add=True)

    def scatter_wait(p):
        pltpu.make_async_copy(rows[p], acc.at[idx[p].at[1]], ssem[p]).wait()

    def item(j, p, first=False, last=False):
        gather_wait(p)
        scatter_start(p)
        if not first:
            scatter_wait((p - 4) % _D)
        if not last:
            idx_start(j + 4, (p + 4) % _D)
            idx_wait((p + 2) % _D)
            gather_start((p + 2) % _D)

    for b in range(4):
        idx_start(b, b)
    for b in range(2):
        idx_wait(b)
        gather_start(b)
    for b in range(_D):
        item(b, b, first=(b < 4))

    @pl.loop(_D, _CH - _D, step=_D)
    def _(i):
        for b in range(_D):
            item(i + b, b)

    for b in range(_D):
        j = _CH - _D + b
        gather_wait(b)
        scatter_start(b)
        scatter_wait((b - 4) % _D)
        if j + 4 < _CH:
            idx_start(j + 4, (b + 4) % _D)
        if j + 2 < _CH:
            idx_wait((b + 2) % _D)
            gather_start((b + 2) % _D)
    for b in range(4, _D):
        scatter_wait(b)

    plsc.subcore_barrier()
    pltpu.sync_copy(acc.at[pl.ds(s * _ZR, _ZR)],
                    out_hbm.at[c, pl.ds(s * _ZR, _ZR)])


_SC_PROP_CACHE = []


def _sc_propagate(table, edges, zeros):
    if not _SC_PROP_CACHE:
        mesh = plsc.VectorSubcoreMesh(core_axis_name="c", subcore_axis_name="s")
        _SC_PROP_CACHE.append(pl.kernel(
            _sc_propagate_body,
            out_type=jax.ShapeDtypeStruct((_NC, _NPAD, _DH), jnp.float32),
            mesh=mesh,
            compiler_params=pltpu.CompilerParams(use_tc_tiling_on_sc=False),
            scratch_types=(
                [pltpu.VMEM((2, _K), jnp.int32)] * _D
                + [pltpu.VMEM((_K, _DH), jnp.float32)] * _D
                + [pltpu.VMEM_SHARED((_NPAD, _DH), jnp.float32)] * 2
                + [pltpu.SemaphoreType.DMA] * (3 * _D)
            ),
        ))
    return _SC_PROP_CACHE[0](table, edges, zeros)


def _mm1_body(x_ref, w_ref, o_ref):
    o_ref[...] = jnp.dot(x_ref[...], w_ref[...],
                         preferred_element_type=jnp.float32)


_tc_mm1 = pl.pallas_call(
    _mm1_body,
    out_shape=jax.ShapeDtypeStruct((_N, _DH), jnp.float32),
)


def _scale_body(degp_ref, h1_ref, tab_ref, dinv_ref):
    dinv = lax.rsqrt(degp_ref[0, 0:_N, :] + degp_ref[1, 0:_N, :] + 1.0)
    dinv_ref[...] = dinv
    tab_ref[...] = jnp.zeros((_NPAD, _DH), jnp.float32)
    tab_ref[0:_N, :] = dinv * h1_ref[...]


_tc_scale = pl.pallas_call(
    _scale_body,
    out_shape=(
        jax.ShapeDtypeStruct((_NPAD, _DH), jnp.float32),
        jax.ShapeDtypeStruct((_N, _DH), jnp.float32),
    ),
)


def _layer1_body(p_ref, u_ref, dinv_ref, b_ref, v_ref):
    p = p_ref[0, 0:_N, :] + p_ref[1, 0:_N, :] + u_ref[0:_N, :]
    h = jnp.maximum(dinv_ref[...] * p + b_ref[...], 0.0)
    v_ref[...] = jnp.zeros((_NPAD, _DH), jnp.float32)
    v_ref[0:_N, :] = dinv_ref[...] * h


_tc_layer1 = pl.pallas_call(
    _layer1_body,
    out_shape=jax.ShapeDtypeStruct((_NPAD, _DH), jnp.float32),
)


def _final_body(p_ref, v_ref, dinv_ref, w2_ref, b2_ref, o_ref):
    g = dinv_ref[...] * (p_ref[0, 0:_N, :] + p_ref[1, 0:_N, :]
                         + v_ref[0:_N, :])
    z = jnp.dot(g, w2_ref[...], preferred_element_type=jnp.float32) + b2_ref[...]
    m = jnp.max(z, axis=1, keepdims=True)
    o_ref[...] = (z - m) - jnp.log(jnp.sum(jnp.exp(z - m), axis=1,
                                           keepdims=True))


_tc_final = pl.pallas_call(
    _final_body,
    out_shape=jax.ShapeDtypeStruct((_N, _DO), jnp.float32),
)


def kernel(x, edge_index, W1, b1, W2, b2):
    src = edge_index[0].astype(jnp.int32)
    dst = edge_index[1].astype(jnp.int32)
    pad = _N + (jnp.arange(_EPAD - _E, dtype=jnp.int32) % (_NPAD - _N))
    src_pad = jnp.concatenate([src, pad]).reshape(_NCH, _K)
    dst_pad = jnp.concatenate([dst, pad]).reshape(_NCH, _K)
    edges = jnp.stack([src_pad, dst_pad], axis=1)
    zeros_tab = jnp.zeros((_NPAD, _DH), jnp.float32)
    ones_tab = jnp.ones((_NPAD, _DH), jnp.float32)

    deg_parts = _sc_propagate(ones_tab, edges, zeros_tab)
    h1 = _tc_mm1(x, W1)
    u_tab, dinv = _tc_scale(deg_parts, h1)
    p1 = _sc_propagate(u_tab, edges, zeros_tab)
    v_tab = _tc_layer1(p1, u_tab, dinv, b1.reshape(1, _DH))
    p2 = _sc_propagate(v_tab, edges, zeros_tab)
    return _tc_final(p2, v_tab, dinv, W2, b2.reshape(1, _DO))


# speedup vs baseline: 56.6591x; 1.0082x over previous
import jax
import jax.numpy as jnp
from jax import lax
from jax.experimental import pallas as pl
from jax.experimental.pallas import tpu as pltpu
from jax.experimental.pallas import tpu_sc as plsc

_N = 10000
_E = 320000
_DF = 128
_DH = 16
_DO = 2

_NC = 2
_NS = 16
_NW = _NC * _NS
_K = 128
_CH = 80
_EPT = _K * _CH
_EPAD = _EPT * _NW
_NCH = _NW * _CH
_NPAD = 10112
_ZR = _NPAD // _NS

_D = 8


def _sc_propagate_body(table_hbm, edges_hbm, zeros_hbm, out_hbm, *refs):
    idx = refs[0:_D]
    rows = refs[_D:2 * _D]
    acc = refs[2 * _D]
    tab = refs[2 * _D + 1]
    isem = refs[2 * _D + 2:2 * _D + 2 + _D]
    gsem = refs[2 * _D + 2 + _D:2 * _D + 2 + 2 * _D]
    ssem = refs[2 * _D + 2 + 2 * _D:2 * _D + 2 + 3 * _D]
    c = lax.axis_index("c")
    s = lax.axis_index("s")
    pltpu.sync_copy(table_hbm.at[pl.ds(s * _ZR, _ZR)],
                    tab.at[pl.ds(s * _ZR, _ZR)])
    pltpu.sync_copy(zeros_hbm.at[pl.ds(s * _ZR, _ZR)],
                    acc.at[pl.ds(s * _ZR, _ZR)])
    plsc.subcore_barrier()
    g0 = (c * _NS + s) * _CH

    def idx_start(j, p):
        pltpu.async_copy(edges_hbm.at[0, g0 + j], idx[p].at[0], isem[p])
        pltpu.async_copy(edges_hbm.at[1, g0 + j], idx[p].at[1], isem[p])

    def idx_wait(p):
        pltpu.make_async_copy(edges_hbm.at[0, g0], idx[p].at[0],
                              isem[p]).wait()
        pltpu.make_async_copy(edges_hbm.at[1, g0], idx[p].at[1],
                              isem[p]).wait()

    def gather_start(p):
        pltpu.async_copy(tab.at[idx[p].at[0]], rows[p], gsem[p])

    def gather_wait(p):
        pltpu.make_async_copy(tab.at[idx[p].at[0]], rows[p],
                              gsem[p]).wait()

    def scatter_start(p):
        pltpu.async_copy(rows[p], acc.at[idx[p].at[1]], ssem[p], add=True)

    def scatter_wait(p):
        pltpu.make_async_copy(rows[p], acc.at[idx[p].at[1]], ssem[p]).wait()

    def item(j, p, first=False, last=False):
        gather_wait(p)
        scatter_start(p)
        if not first:
            scatter_wait((p - 4) % _D)
        if not last:
            idx_start(j + 4, (p + 4) % _D)
            idx_wait((p + 2) % _D)
            gather_start((p + 2) % _D)

    for b in range(4):
        idx_start(b, b)
    for b in range(2):
        idx_wait(b)
        gather_start(b)
    for b in range(_D):
        item(b, b, first=(b < 4))

    @pl.loop(_D, _CH - _D, step=_D)
    def _(i):
        for b in range(_D):
            item(i + b, b)

    for b in range(_D):
        j = _CH - _D + b
        gather_wait(b)
        scatter_start(b)
        scatter_wait((b - 4) % _D)
        if j + 4 < _CH:
            idx_start(j + 4, (b + 4) % _D)
        if j + 2 < _CH:
            idx_wait((b + 2) % _D)
            gather_start((b + 2) % _D)
    for b in range(4, _D):
        scatter_wait(b)

    plsc.subcore_barrier()
    pltpu.sync_copy(acc.at[pl.ds(s * _ZR, _ZR)],
                    out_hbm.at[c, pl.ds(s * _ZR, _ZR)])


_SC_PROP_CACHE = []


def _sc_propagate(table, edges, zeros):
    if not _SC_PROP_CACHE:
        mesh = plsc.VectorSubcoreMesh(core_axis_name="c", subcore_axis_name="s")
        _SC_PROP_CACHE.append(pl.kernel(
            _sc_propagate_body,
            out_type=jax.ShapeDtypeStruct((_NC, _NPAD, _DH), jnp.float32),
            mesh=mesh,
            compiler_params=pltpu.CompilerParams(use_tc_tiling_on_sc=False),
            scratch_types=(
                [pltpu.VMEM((2, _K), jnp.int32)] * _D
                + [pltpu.VMEM((_K, _DH), jnp.float32)] * _D
                + [pltpu.VMEM_SHARED((_NPAD, _DH), jnp.float32)] * 2
                + [pltpu.SemaphoreType.DMA] * (3 * _D)
            ),
        ))
    return _SC_PROP_CACHE[0](table, edges, zeros)


def _mm1_body(x_ref, w_ref, o_ref):
    o_ref[...] = jnp.dot(x_ref[...], w_ref[...],
                         preferred_element_type=jnp.float32)


_tc_mm1 = pl.pallas_call(
    _mm1_body,
    out_shape=jax.ShapeDtypeStruct((_N, _DH), jnp.float32),
)


def _scale_body(degp_ref, h1_ref, tab_ref, dinv_ref):
    dinv = lax.rsqrt(degp_ref[0, 0:_N, :] + degp_ref[1, 0:_N, :] + 1.0)
    dinv_ref[...] = dinv
    tab_ref[...] = jnp.zeros((_NPAD, _DH), jnp.float32)
    tab_ref[0:_N, :] = dinv * h1_ref[...]


_tc_scale = pl.pallas_call(
    _scale_body,
    out_shape=(
        jax.ShapeDtypeStruct((_NPAD, _DH), jnp.float32),
        jax.ShapeDtypeStruct((_N, _DH), jnp.float32),
    ),
)


def _layer1_body(p_ref, u_ref, dinv_ref, b_ref, v_ref):
    p = p_ref[0, 0:_N, :] + p_ref[1, 0:_N, :] + u_ref[0:_N, :]
    h = jnp.maximum(dinv_ref[...] * p + b_ref[...], 0.0)
    v_ref[...] = jnp.zeros((_NPAD, _DH), jnp.float32)
    v_ref[0:_N, :] = dinv_ref[...] * h


_tc_layer1 = pl.pallas_call(
    _layer1_body,
    out_shape=jax.ShapeDtypeStruct((_NPAD, _DH), jnp.float32),
)


def _final_body(p_ref, v_ref, dinv_ref, w2_ref, b2_ref, o_ref):
    g = dinv_ref[...] * (p_ref[0, 0:_N, :] + p_ref[1, 0:_N, :]
                         + v_ref[0:_N, :])
    z = jnp.dot(g, w2_ref[...], preferred_element_type=jnp.float32) + b2_ref[...]
    m = jnp.max(z, axis=1, keepdims=True)
    o_ref[...] = (z - m) - jnp.log(jnp.sum(jnp.exp(z - m), axis=1,
                                           keepdims=True))


_tc_final = pl.pallas_call(
    _final_body,
    out_shape=jax.ShapeDtypeStruct((_N, _DO), jnp.float32),
)


def kernel(x, edge_index, W1, b1, W2, b2):
    src = edge_index[0].astype(jnp.int32)
    dst = edge_index[1].astype(jnp.int32)
    pad = _N + (jnp.arange(_EPAD - _E, dtype=jnp.int32) % (_NPAD - _N))
    edges = jnp.stack([jnp.concatenate([src, pad]),
                       jnp.concatenate([dst, pad])]).reshape(2, _NCH, _K)
    zeros_tab = jnp.zeros((_NPAD, _DH), jnp.float32)
    ones_tab = jnp.ones((_NPAD, _DH), jnp.float32)

    deg_parts = _sc_propagate(ones_tab, edges, zeros_tab)
    h1 = _tc_mm1(x, W1)
    u_tab, dinv = _tc_scale(deg_parts, h1)
    p1 = _sc_propagate(u_tab, edges, zeros_tab)
    v_tab = _tc_layer1(p1, u_tab, dinv, b1.reshape(1, _DH))
    p2 = _sc_propagate(v_tab, edges, zeros_tab)
    return _tc_final(p2, v_tab, dinv, W2, b2.reshape(1, _DO))


# speedup vs baseline: 60.3423x; 1.0650x over previous
import jax
import jax.numpy as jnp
from jax import lax
from jax.experimental import pallas as pl
from jax.experimental.pallas import tpu as pltpu
from jax.experimental.pallas import tpu_sc as plsc

_N = 10000
_E = 320000
_DF = 128
_DH = 16
_DO = 2

_NC = 2
_NS = 16
_NW = _NC * _NS
_K = 128
_CH = 80
_EPT = _K * _CH
_EPAD = _EPT * _NW
_NCH = _NW * _CH
_NPAD = 10112
_ZR = _NPAD // _NS

_D = 8


_NCH_REAL = _E // _K


def _sc_propagate_body(table_hbm, edges_hbm, pad_edges_hbm, zeros_hbm,
                       out_hbm, *refs):
    idx = refs[0:_D]
    rows = refs[_D:2 * _D]
    acc = refs[2 * _D]
    tab = refs[2 * _D + 1]
    isem = refs[2 * _D + 2:2 * _D + 2 + _D]
    gsem = refs[2 * _D + 2 + _D:2 * _D + 2 + 2 * _D]
    ssem = refs[2 * _D + 2 + 2 * _D:2 * _D + 2 + 3 * _D]
    c = lax.axis_index("c")
    s = lax.axis_index("s")
    pltpu.sync_copy(table_hbm.at[pl.ds(s * _ZR, _ZR)],
                    tab.at[pl.ds(s * _ZR, _ZR)])
    pltpu.sync_copy(zeros_hbm.at[pl.ds(s * _ZR, _ZR)],
                    acc.at[pl.ds(s * _ZR, _ZR)])
    plsc.subcore_barrier()
    g0 = (c * _NS + s) * _CH

    def idx_start(j, p):
        g = g0 + j

        @pl.when(g < _NCH_REAL)
        def _():
            pltpu.async_copy(edges_hbm.at[0, g], idx[p].at[0], isem[p])
            pltpu.async_copy(edges_hbm.at[1, g], idx[p].at[1], isem[p])

        @pl.when(g >= _NCH_REAL)
        def _():
            pltpu.async_copy(pad_edges_hbm.at[0, g - _NCH_REAL],
                             idx[p].at[0], isem[p])
            pltpu.async_copy(pad_edges_hbm.at[1, g - _NCH_REAL],
                             idx[p].at[1], isem[p])

    def idx_wait(p):
        pltpu.make_async_copy(edges_hbm.at[0, g0], idx[p].at[0],
                              isem[p]).wait()
        pltpu.make_async_copy(edges_hbm.at[1, g0], idx[p].at[1],
                              isem[p]).wait()

    def gather_start(p):
        pltpu.async_copy(tab.at[idx[p].at[0]], rows[p], gsem[p])

    def gather_wait(p):
        pltpu.make_async_copy(tab.at[idx[p].at[0]], rows[p],
                              gsem[p]).wait()

    def scatter_start(p):
        pltpu.async_copy(rows[p], acc.at[idx[p].at[1]], ssem[p], add=True)

    def scatter_wait(p):
        pltpu.make_async_copy(rows[p], acc.at[idx[p].at[1]], ssem[p]).wait()

    def item(j, p, first=False, last=False):
        gather_wait(p)
        scatter_start(p)
        if not first:
            scatter_wait((p - 4) % _D)
        if not last:
            idx_start(j + 4, (p + 4) % _D)
            idx_wait((p + 2) % _D)
            gather_start((p + 2) % _D)

    for b in range(4):
        idx_start(b, b)
    for b in range(2):
        idx_wait(b)
        gather_start(b)
    for b in range(_D):
        item(b, b, first=(b < 4))

    @pl.loop(_D, _CH - _D, step=_D)
    def _(i):
        for b in range(_D):
            item(i + b, b)

    for b in range(_D):
        j = _CH - _D + b
        gather_wait(b)
        scatter_start(b)
        scatter_wait((b - 4) % _D)
        if j + 4 < _CH:
            idx_start(j + 4, (b + 4) % _D)
        if j + 2 < _CH:
            idx_wait((b + 2) % _D)
            gather_start((b + 2) % _D)
    for b in range(4, _D):
        scatter_wait(b)

    plsc.subcore_barrier()
    pltpu.sync_copy(acc.at[pl.ds(s * _ZR, _ZR)],
                    out_hbm.at[c, pl.ds(s * _ZR, _ZR)])


_SC_PROP_CACHE = []


def _sc_propagate(table, edges, pad_edges, zeros):
    if not _SC_PROP_CACHE:
        mesh = plsc.VectorSubcoreMesh(core_axis_name="c", subcore_axis_name="s")
        _SC_PROP_CACHE.append(pl.kernel(
            _sc_propagate_body,
            out_type=jax.ShapeDtypeStruct((_NC, _NPAD, _DH), jnp.float32),
            mesh=mesh,
            compiler_params=pltpu.CompilerParams(use_tc_tiling_on_sc=False),
            scratch_types=(
                [pltpu.VMEM((2, _K), jnp.int32)] * _D
                + [pltpu.VMEM((_K, _DH), jnp.float32)] * _D
                + [pltpu.VMEM_SHARED((_NPAD, _DH), jnp.float32)] * 2
                + [pltpu.SemaphoreType.DMA] * (3 * _D)
            ),
        ))
    return _SC_PROP_CACHE[0](table, edges, pad_edges, zeros)


def _mm1_body(x_ref, w_ref, o_ref):
    o_ref[...] = jnp.dot(x_ref[...], w_ref[...],
                         preferred_element_type=jnp.float32)


_tc_mm1 = pl.pallas_call(
    _mm1_body,
    out_shape=jax.ShapeDtypeStruct((_N, _DH), jnp.float32),
)


def _scale_body(degp_ref, h1_ref, tab_ref, dinv_ref):
    dinv = lax.rsqrt(degp_ref[0, 0:_N, :] + degp_ref[1, 0:_N, :] + 1.0)
    dinv_ref[...] = dinv
    tab_ref[...] = jnp.zeros((_NPAD, _DH), jnp.float32)
    tab_ref[0:_N, :] = dinv * h1_ref[...]


_tc_scale = pl.pallas_call(
    _scale_body,
    out_shape=(
        jax.ShapeDtypeStruct((_NPAD, _DH), jnp.float32),
        jax.ShapeDtypeStruct((_N, _DH), jnp.float32),
    ),
)


def _layer1_body(p_ref, u_ref, dinv_ref, b_ref, v_ref):
    p = p_ref[0, 0:_N, :] + p_ref[1, 0:_N, :] + u_ref[0:_N, :]
    h = jnp.maximum(dinv_ref[...] * p + b_ref[...], 0.0)
    v_ref[...] = jnp.zeros((_NPAD, _DH), jnp.float32)
    v_ref[0:_N, :] = dinv_ref[...] * h


_tc_layer1 = pl.pallas_call(
    _layer1_body,
    out_shape=jax.ShapeDtypeStruct((_NPAD, _DH), jnp.float32),
)


def _final_body(p_ref, v_ref, dinv_ref, w2_ref, b2_ref, o_ref):
    g = dinv_ref[...] * (p_ref[0, 0:_N, :] + p_ref[1, 0:_N, :]
                         + v_ref[0:_N, :])
    z = jnp.dot(g, w2_ref[...], preferred_element_type=jnp.float32) + b2_ref[...]
    m = jnp.max(z, axis=1, keepdims=True)
    o_ref[...] = (z - m) - jnp.log(jnp.sum(jnp.exp(z - m), axis=1,
                                           keepdims=True))


_tc_final = pl.pallas_call(
    _final_body,
    out_shape=jax.ShapeDtypeStruct((_N, _DO), jnp.float32),
)


def kernel(x, edge_index, W1, b1, W2, b2):
    edges = edge_index.astype(jnp.int32).reshape(2, _NCH_REAL, _K)
    pad = _N + (jnp.arange(_EPAD - _E, dtype=jnp.int32) % (_NPAD - _N))
    pad_edges = jnp.broadcast_to(
        pad.reshape(1, _NCH - _NCH_REAL, _K), (2, _NCH - _NCH_REAL, _K))
    zeros_tab = jnp.zeros((_NPAD, _DH), jnp.float32)
    ones_tab = jnp.ones((_NPAD, _DH), jnp.float32)

    deg_parts = _sc_propagate(ones_tab, edges, pad_edges, zeros_tab)
    h1 = _tc_mm1(x, W1)
    u_tab, dinv = _tc_scale(deg_parts, h1)
    p1 = _sc_propagate(u_tab, edges, pad_edges, zeros_tab)
    v_tab = _tc_layer1(p1, u_tab, dinv, b1.reshape(1, _DH))
    p2 = _sc_propagate(v_tab, edges, pad_edges, zeros_tab)
    return _tc_final(p2, v_tab, dinv, W2, b2.reshape(1, _DO))


# speedup vs baseline: 63.3203x; 1.0494x over previous
import jax
import jax.numpy as jnp
from jax import lax
from jax.experimental import pallas as pl
from jax.experimental.pallas import tpu as pltpu
from jax.experimental.pallas import tpu_sc as plsc

_N = 10000
_E = 320000
_DF = 128
_DH = 16
_DO = 2

_NC = 2
_NS = 16
_NW = _NC * _NS
_K = 128
_CH = 80
_EPT = _K * _CH
_EPAD = _EPT * _NW
_NCH = _NW * _CH
_NPAD = 10112
_ZR = _NPAD // _NS

_D = 8


_NCH_REAL = _E // _K


def _sc_propagate_body(table_hbm, edges_hbm, pad_edges_hbm, zeros_hbm,
                       out_hbm, *refs):
    _sc_pass(True, table_hbm, edges_hbm, pad_edges_hbm, zeros_hbm, out_hbm,
             refs)


def _sc_degree_body(edges_hbm, pad_edges_hbm, zeros_hbm, out_hbm, *refs):
    _sc_pass(False, None, edges_hbm, pad_edges_hbm, zeros_hbm, out_hbm, refs)


def _sc_pass(with_gather, table_hbm, edges_hbm, pad_edges_hbm, zeros_hbm,
             out_hbm, refs):
    idx = refs[0:_D]
    if with_gather:
        rows = refs[_D:2 * _D]
        acc = refs[2 * _D]
        tab = refs[2 * _D + 1]
        sem0 = 2 * _D + 2
        gsem = refs[sem0 + _D:sem0 + 2 * _D]
    else:
        ones = refs[_D]
        acc = refs[_D + 1]
        sem0 = _D + 2
    isem = refs[sem0:sem0 + _D]
    ssem = refs[sem0 + 2 * _D:sem0 + 3 * _D] if with_gather \
        else refs[sem0 + _D:sem0 + 2 * _D]
    c = lax.axis_index("c")
    s = lax.axis_index("s")
    if with_gather:
        pltpu.sync_copy(table_hbm.at[pl.ds(s * _ZR, _ZR)],
                        tab.at[pl.ds(s * _ZR, _ZR)])
    else:
        @pl.loop(0, _K)
        def _(i):
            ones.at[i][...] = jnp.full((_DH,), 1.0, jnp.float32)
    pltpu.sync_copy(zeros_hbm.at[pl.ds(s * _ZR, _ZR)],
                    acc.at[pl.ds(s * _ZR, _ZR)])
    plsc.subcore_barrier()
    g0 = (c * _NS + s) * _CH

    def idx_start(j, p):
        g = g0 + j

        @pl.when(g < _NCH_REAL)
        def _():
            pltpu.async_copy(edges_hbm.at[0, g], idx[p].at[0], isem[p])
            pltpu.async_copy(edges_hbm.at[1, g], idx[p].at[1], isem[p])

        @pl.when(g >= _NCH_REAL)
        def _():
            pltpu.async_copy(pad_edges_hbm.at[0, g - _NCH_REAL],
                             idx[p].at[0], isem[p])
            pltpu.async_copy(pad_edges_hbm.at[1, g - _NCH_REAL],
                             idx[p].at[1], isem[p])

    def idx_wait(p):
        pltpu.make_async_copy(edges_hbm.at[0, g0], idx[p].at[0],
                              isem[p]).wait()
        pltpu.make_async_copy(edges_hbm.at[1, g0], idx[p].at[1],
                              isem[p]).wait()

    def gather_start(p):
        if with_gather:
            pltpu.async_copy(tab.at[idx[p].at[0]], rows[p], gsem[p])

    def gather_wait(p):
        if with_gather:
            pltpu.make_async_copy(tab.at[idx[p].at[0]], rows[p],
                                  gsem[p]).wait()

    def scatter_start(p):
        src = rows[p] if with_gather else ones
        pltpu.async_copy(src, acc.at[idx[p].at[1]], ssem[p], add=True)

    def scatter_wait(p):
        src = rows[p] if with_gather else ones
        pltpu.make_async_copy(src, acc.at[idx[p].at[1]], ssem[p]).wait()

    def item(j, p, first=False, last=False):
        gather_wait(p)
        if not with_gather:
            idx_wait(p)
        scatter_start(p)
        if not first:
            scatter_wait((p - 4) % _D)
        if not last:
            idx_start(j + 4, (p + 4) % _D)
            if with_gather:
                idx_wait((p + 2) % _D)
                gather_start((p + 2) % _D)

    for b in range(4):
        idx_start(b, b)
    if with_gather:
        for b in range(2):
            idx_wait(b)
            gather_start(b)
    for b in range(_D):
        item(b, b, first=(b < 4))

    @pl.loop(_D, _CH - _D, step=_D)
    def _(i):
        for b in range(_D):
            item(i + b, b)

    for b in range(_D):
        j = _CH - _D + b
        gather_wait(b)
        if not with_gather:
            idx_wait(b)
        scatter_start(b)
        scatter_wait((b - 4) % _D)
        if j + 4 < _CH:
            idx_start(j + 4, (b + 4) % _D)
        if with_gather and j + 2 < _CH:
            idx_wait((b + 2) % _D)
            gather_start((b + 2) % _D)
    for b in range(4, _D):
        scatter_wait(b)

    plsc.subcore_barrier()
    pltpu.sync_copy(acc.at[pl.ds(s * _ZR, _ZR)],
                    out_hbm.at[c, pl.ds(s * _ZR, _ZR)])


_SC_CACHE = {}


def _sc_propagate(table, edges, pad_edges, zeros):
    if "prop" not in _SC_CACHE:
        mesh = plsc.VectorSubcoreMesh(core_axis_name="c", subcore_axis_name="s")
        _SC_CACHE["prop"] = pl.kernel(
            _sc_propagate_body,
            out_type=jax.ShapeDtypeStruct((_NC, _NPAD, _DH), jnp.float32),
            mesh=mesh,
            compiler_params=pltpu.CompilerParams(use_tc_tiling_on_sc=False),
            scratch_types=(
                [pltpu.VMEM((2, _K), jnp.int32)] * _D
                + [pltpu.VMEM((_K, _DH), jnp.float32)] * _D
                + [pltpu.VMEM_SHARED((_NPAD, _DH), jnp.float32)] * 2
                + [pltpu.SemaphoreType.DMA] * (3 * _D)
            ),
        )
    return _SC_CACHE["prop"](table, edges, pad_edges, zeros)


def _sc_degree(edges, pad_edges, zeros):
    if "deg" not in _SC_CACHE:
        mesh = plsc.VectorSubcoreMesh(core_axis_name="c", subcore_axis_name="s")
        _SC_CACHE["deg"] = pl.kernel(
            _sc_degree_body,
            out_type=jax.ShapeDtypeStruct((_NC, _NPAD, _DH), jnp.float32),
            mesh=mesh,
            compiler_params=pltpu.CompilerParams(use_tc_tiling_on_sc=False),
            scratch_types=(
                [pltpu.VMEM((2, _K), jnp.int32)] * _D
                + [pltpu.VMEM((_K, _DH), jnp.float32)]
                + [pltpu.VMEM_SHARED((_NPAD, _DH), jnp.float32)]
                + [pltpu.SemaphoreType.DMA] * (2 * _D)
            ),
        )
    return _SC_CACHE["deg"](edges, pad_edges, zeros)


def _mm1_body(x_ref, w_ref, o_ref):
    o_ref[...] = jnp.dot(x_ref[...], w_ref[...],
                         preferred_element_type=jnp.float32)


_tc_mm1 = pl.pallas_call(
    _mm1_body,
    out_shape=jax.ShapeDtypeStruct((_N, _DH), jnp.float32),
)


def _scale_body(degp_ref, h1_ref, tab_ref, dinv_ref):
    dinv = lax.rsqrt(degp_ref[0, 0:_N, :] + degp_ref[1, 0:_N, :] + 1.0)
    dinv_ref[...] = dinv
    tab_ref[...] = jnp.zeros((_NPAD, _DH), jnp.float32)
    tab_ref[0:_N, :] = dinv * h1_ref[...]


_tc_scale = pl.pallas_call(
    _scale_body,
    out_shape=(
        jax.ShapeDtypeStruct((_NPAD, _DH), jnp.float32),
        jax.ShapeDtypeStruct((_N, _DH), jnp.float32),
    ),
)


def _layer1_body(p_ref, u_ref, dinv_ref, b_ref, v_ref):
    p = p_ref[0, 0:_N, :] + p_ref[1, 0:_N, :] + u_ref[0:_N, :]
    h = jnp.maximum(dinv_ref[...] * p + b_ref[...], 0.0)
    v_ref[...] = jnp.zeros((_NPAD, _DH), jnp.float32)
    v_ref[0:_N, :] = dinv_ref[...] * h


_tc_layer1 = pl.pallas_call(
    _layer1_body,
    out_shape=jax.ShapeDtypeStruct((_NPAD, _DH), jnp.float32),
)


def _final_body(p_ref, v_ref, dinv_ref, w2_ref, b2_ref, o_ref):
    g = dinv_ref[...] * (p_ref[0, 0:_N, :] + p_ref[1, 0:_N, :]
                         + v_ref[0:_N, :])
    z = jnp.dot(g, w2_ref[...], preferred_element_type=jnp.float32) + b2_ref[...]
    m = jnp.max(z, axis=1, keepdims=True)
    o_ref[...] = (z - m) - jnp.log(jnp.sum(jnp.exp(z - m), axis=1,
                                           keepdims=True))


_tc_final = pl.pallas_call(
    _final_body,
    out_shape=jax.ShapeDtypeStruct((_N, _DO), jnp.float32),
)


def kernel(x, edge_index, W1, b1, W2, b2):
    edges = edge_index.astype(jnp.int32).reshape(2, _NCH_REAL, _K)
    pad = _N + (jnp.arange(_EPAD - _E, dtype=jnp.int32) % (_NPAD - _N))
    pad_edges = jnp.broadcast_to(
        pad.reshape(1, _NCH - _NCH_REAL, _K), (2, _NCH - _NCH_REAL, _K))
    zeros_tab = jnp.zeros((_NPAD, _DH), jnp.float32)

    deg_parts = _sc_degree(edges, pad_edges, zeros_tab)
    h1 = _tc_mm1(x, W1)
    u_tab, dinv = _tc_scale(deg_parts, h1)
    p1 = _sc_propagate(u_tab, edges, pad_edges, zeros_tab)
    v_tab = _tc_layer1(p1, u_tab, dinv, b1.reshape(1, _DH))
    p2 = _sc_propagate(v_tab, edges, pad_edges, zeros_tab)
    return _tc_final(p2, v_tab, dinv, W2, b2.reshape(1, _DO))


# speedup vs baseline: 73.9402x; 1.1677x over previous
import jax
import jax.numpy as jnp
from jax import lax
from jax.experimental import pallas as pl
from jax.experimental.pallas import tpu as pltpu
from jax.experimental.pallas import tpu_sc as plsc

_N = 10000
_E = 320000
_DF = 128
_DH = 16
_DO = 2

_NC = 2
_NS = 16
_NW = _NC * _NS
_K = 128
_CH = 80
_EPT = _K * _CH
_EPAD = _EPT * _NW
_NCH = _NW * _CH
_NCH_REAL = _E // _K
_NPAD = 10112
_ZR = _NPAD // _NS
_D = 8


def _rsqrt16(d):
    i = lax.bitcast_convert_type(d, jnp.int32)
    i = jnp.full((_DH,), 0x5F3759DF, jnp.int32) - lax.shift_right_logical(
        i, jnp.full((_DH,), 1, jnp.int32))
    y = lax.bitcast_convert_type(i, jnp.float32)
    for _ in range(3):
        y = y * (1.5 - 0.5 * d * y * y)
    return y


def _edge_pass(with_gather, edges_hbm, pad_edges_hbm, out_hbm, idx,
               rows_or_ones, acc, tab, isem, gsem, ssem, c, s):
    plsc.subcore_barrier()
    g0 = (c * _NS + s) * _CH

    def idx_start(j, p):
        g = g0 + j

        @pl.when(g < _NCH_REAL)
        def _():
            pltpu.async_copy(edges_hbm.at[0, g], idx[p].at[0], isem[p])
            pltpu.async_copy(edges_hbm.at[1, g], idx[p].at[1], isem[p])

        @pl.when(g >= _NCH_REAL)
        def _():
            pltpu.async_copy(pad_edges_hbm.at[0, g - _NCH_REAL],
                             idx[p].at[0], isem[p])
            pltpu.async_copy(pad_edges_hbm.at[1, g - _NCH_REAL],
                             idx[p].at[1], isem[p])

    def idx_wait(p):
        pltpu.make_async_copy(edges_hbm.at[0, g0], idx[p].at[0],
                              isem[p]).wait()
        pltpu.make_async_copy(edges_hbm.at[1, g0], idx[p].at[1],
                              isem[p]).wait()

    def gather_start(p):
        if with_gather:
            pltpu.async_copy(tab.at[idx[p].at[0]], rows_or_ones[p], gsem[p])

    def gather_wait(p):
        if with_gather:
            pltpu.make_async_copy(tab.at[idx[p].at[0]], rows_or_ones[p],
                                  gsem[p]).wait()

    def scatter_start(p):
        src = rows_or_ones[p] if with_gather else rows_or_ones
        pltpu.async_copy(src, acc.at[idx[p].at[1]], ssem[p], add=True)

    def scatter_wait(p):
        src = rows_or_ones[p] if with_gather else rows_or_ones
        pltpu.make_async_copy(src, acc.at[idx[p].at[1]], ssem[p]).wait()

    def item(j, p, first=False):
        gather_wait(p)
        if not with_gather:
            idx_wait(p)
        scatter_start(p)
        if not first:
            scatter_wait((p - 4) % _D)
        idx_start(j + 4, (p + 4) % _D)
        if with_gather:
            idx_wait((p + 2) % _D)
            gather_start((p + 2) % _D)

    for b in range(4):
        idx_start(b, b)
    if with_gather:
        for b in range(2):
            idx_wait(b)
            gather_start(b)
    for b in range(_D):
        item(b, b, first=(b < 4))

    @pl.loop(_D, _CH - _D, step=_D)
    def _(i):
        for b in range(_D):
            item(i + b, b)

    for b in range(_D):
        j = _CH - _D + b
        gather_wait(b)
        if not with_gather:
            idx_wait(b)
        scatter_start(b)
        scatter_wait((b - 4) % _D)
        if j + 4 < _CH:
            idx_start(j + 4, (b + 4) % _D)
        if with_gather and j + 2 < _CH:
            idx_wait((b + 2) % _D)
            gather_start((b + 2) % _D)
    for b in range(4, _D):
        scatter_wait(b)

    plsc.subcore_barrier()
    pltpu.sync_copy(acc.at[pl.ds(s * _ZR, _ZR)],
                    out_hbm.at[c, pl.ds(s * _ZR, _ZR)])


def _sc_degree_body(edges_hbm, pad_edges_hbm, zeros_hbm, out_hbm, *refs):
    idx = refs[0:_D]
    ones = refs[_D]
    acc = refs[_D + 1]
    isem = refs[_D + 2:2 * _D + 2]
    ssem = refs[2 * _D + 2:3 * _D + 2]
    c = lax.axis_index("c")
    s = lax.axis_index("s")

    @pl.loop(0, _K)
    def _(i):
        ones.at[i][...] = jnp.full((_DH,), 1.0, jnp.float32)

    pltpu.sync_copy(zeros_hbm.at[pl.ds(s * _ZR, _ZR)],
                    acc.at[pl.ds(s * _ZR, _ZR)])
    _edge_pass(False, edges_hbm, pad_edges_hbm, out_hbm, idx, ones, acc,
               None, isem, None, ssem, c, s)


def _sc_pass1_body(h_hbm, degp_hbm, edges_hbm, pad_edges_hbm, zeros_hbm,
                   outp_hbm, dinv_hbm, u_hbm, *refs):
    idx = refs[0:_D]
    rows = refs[_D:2 * _D]
    acc, tab = refs[2 * _D], refs[2 * _D + 1]
    d0b, d1b, hb, yb, ub = refs[2 * _D + 2:2 * _D + 7]
    sem0 = 2 * _D + 7
    isem = refs[sem0:sem0 + _D]
    gsem = refs[sem0 + _D:sem0 + 2 * _D]
    ssem = refs[sem0 + 2 * _D:sem0 + 3 * _D]
    c = lax.axis_index("c")
    s = lax.axis_index("s")
    r0 = s * _ZR
    pltpu.sync_copy(degp_hbm.at[0, pl.ds(r0, _ZR)], d0b)
    pltpu.sync_copy(degp_hbm.at[1, pl.ds(r0, _ZR)], d1b)
    pltpu.sync_copy(h_hbm.at[pl.ds(r0, _ZR)], hb)
    pltpu.sync_copy(zeros_hbm.at[pl.ds(r0, _ZR)], acc.at[pl.ds(r0, _ZR)])

    @pl.loop(0, _ZR)
    def _(i):
        d = d0b.at[i][...] + d1b.at[i][...] + 1.0
        y = _rsqrt16(d)
        yb.at[i][...] = y
        ub.at[i][...] = y * hb.at[i][...]

    pltpu.sync_copy(ub, tab.at[pl.ds(r0, _ZR)])

    @pl.when(c == 0)
    def _():
        pltpu.sync_copy(yb, dinv_hbm.at[pl.ds(r0, _ZR)])
        pltpu.sync_copy(ub, u_hbm.at[pl.ds(r0, _ZR)])

    _edge_pass(True, edges_hbm, pad_edges_hbm, outp_hbm, idx, rows, acc,
               tab, isem, gsem, ssem, c, s)


def _sc_pass2_body(dinv_hbm, u_hbm, p1p_hbm, b1_hbm, edges_hbm,
                   pad_edges_hbm, zeros_hbm, outp_hbm, v_hbm, *refs):
    idx = refs[0:_D]
    rows = refs[_D:2 * _D]
    acc, tab = refs[2 * _D], refs[2 * _D + 1]
    yb, ub, p0b, p1b, vb, bb = refs[2 * _D + 2:2 * _D + 8]
    sem0 = 2 * _D + 8
    isem = refs[sem0:sem0 + _D]
    gsem = refs[sem0 + _D:sem0 + 2 * _D]
    ssem = refs[sem0 + 2 * _D:sem0 + 3 * _D]
    c = lax.axis_index("c")
    s = lax.axis_index("s")
    r0 = s * _ZR
    pltpu.sync_copy(dinv_hbm.at[pl.ds(r0, _ZR)], yb)
    pltpu.sync_copy(u_hbm.at[pl.ds(r0, _ZR)], ub)
    pltpu.sync_copy(p1p_hbm.at[0, pl.ds(r0, _ZR)], p0b)
    pltpu.sync_copy(p1p_hbm.at[1, pl.ds(r0, _ZR)], p1b)
    pltpu.sync_copy(b1_hbm, bb)
    pltpu.sync_copy(zeros_hbm.at[pl.ds(r0, _ZR)], acc.at[pl.ds(r0, _ZR)])

    @pl.loop(0, _ZR)
    def _(i):
        y = yb.at[i][...]
        p = p0b.at[i][...] + p1b.at[i][...] + ub.at[i][...]
        hh = jnp.maximum(y * p + bb[...], 0.0)
        vb.at[i][...] = y * hh

    pltpu.sync_copy(vb, tab.at[pl.ds(r0, _ZR)])

    @pl.when(c == 0)
    def _():
        pltpu.sync_copy(vb, v_hbm.at[pl.ds(r0, _ZR)])

    _edge_pass(True, edges_hbm, pad_edges_hbm, outp_hbm, idx, rows, acc,
               tab, isem, gsem, ssem, c, s)


_SC_CACHE = {}
_PARTS = jax.ShapeDtypeStruct((_NC, _NPAD, _DH), jnp.float32)
_TAB = jax.ShapeDtypeStruct((_NPAD, _DH), jnp.float32)
_IDX_RING = [pltpu.VMEM((2, _K), jnp.int32)] * _D
_ROW_RING = [pltpu.VMEM((_K, _DH), jnp.float32)] * _D
_ZRBUF = pltpu.VMEM((_ZR, _DH), jnp.float32)
_SPMEM = pltpu.VMEM_SHARED((_NPAD, _DH), jnp.float32)


def _get_sc(name):
    if name not in _SC_CACHE:
        mesh = plsc.VectorSubcoreMesh(core_axis_name="c", subcore_axis_name="s")
        cp = pltpu.CompilerParams(use_tc_tiling_on_sc=False)
        if name == "deg":
            _SC_CACHE[name] = pl.kernel(
                _sc_degree_body, out_type=_PARTS, mesh=mesh,
                compiler_params=cp,
                scratch_types=(
                    _IDX_RING
                    + [pltpu.VMEM((_K, _DH), jnp.float32), _SPMEM]
                    + [pltpu.SemaphoreType.DMA] * (2 * _D)
                ),
            )
        elif name == "pass1":
            _SC_CACHE[name] = pl.kernel(
                _sc_pass1_body, out_type=(_PARTS, _TAB, _TAB), mesh=mesh,
                compiler_params=cp,
                scratch_types=(
                    _IDX_RING + _ROW_RING + [_SPMEM, _SPMEM]
                    + [_ZRBUF] * 5
                    + [pltpu.SemaphoreType.DMA] * (3 * _D)
                ),
            )
        else:
            _SC_CACHE[name] = pl.kernel(
                _sc_pass2_body, out_type=(_PARTS, _TAB), mesh=mesh,
                compiler_params=cp,
                scratch_types=(
                    _IDX_RING + _ROW_RING + [_SPMEM, _SPMEM]
                    + [_ZRBUF] * 5 + [pltpu.VMEM((_DH,), jnp.float32)]
                    + [pltpu.SemaphoreType.DMA] * (3 * _D)
                ),
            )
    return _SC_CACHE[name]


def _mm1_body(x_ref, w_ref, o_ref):
    o_ref[...] = jnp.zeros((_NPAD, _DH), jnp.float32)
    o_ref[0:_N, :] = jnp.dot(x_ref[...], w_ref[...],
                             preferred_element_type=jnp.float32)


_tc_mm1 = pl.pallas_call(
    _mm1_body,
    out_shape=jax.ShapeDtypeStruct((_NPAD, _DH), jnp.float32),
)


def _final_body(p_ref, v_ref, dinv_ref, w2_ref, b2_ref, o_ref):
    g = dinv_ref[0:_N, :] * (p_ref[0, 0:_N, :] + p_ref[1, 0:_N, :]
                             + v_ref[0:_N, :])
    z = jnp.dot(g, w2_ref[...], preferred_element_type=jnp.float32) + b2_ref[...]
    m = jnp.max(z, axis=1, keepdims=True)
    o_ref[...] = (z - m) - jnp.log(jnp.sum(jnp.exp(z - m), axis=1,
                                           keepdims=True))


_tc_final = pl.pallas_call(
    _final_body,
    out_shape=jax.ShapeDtypeStruct((_N, _DO), jnp.float32),
)


def kernel(x, edge_index, W1, b1, W2, b2):
    edges = edge_index.astype(jnp.int32).reshape(2, _NCH_REAL, _K)
    pad = _N + (jnp.arange(_EPAD - _E, dtype=jnp.int32) % (_NPAD - _N))
    pad_edges = jnp.broadcast_to(
        pad.reshape(1, _NCH - _NCH_REAL, _K), (2, _NCH - _NCH_REAL, _K))
    zeros_tab = jnp.zeros((_NPAD, _DH), jnp.float32)

    deg_parts = _get_sc("deg")(edges, pad_edges, zeros_tab)
    h_tab = _tc_mm1(x, W1)
    p1, dinv_tab, u_tab = _get_sc("pass1")(h_tab, deg_parts, edges,
                                           pad_edges, zeros_tab)
    p2, v_tab = _get_sc("pass2")(dinv_tab, u_tab, p1, b1, edges, pad_edges,
                                 zeros_tab)
    return _tc_final(p2, v_tab, dinv_tab, W2, b2.reshape(1, _DO))
